# Initial kernel scaffold; baseline (speedup 1.0000x reference)
import jax
import jax.numpy as jnp
from jax.experimental import pallas as pl


def kernel(node_type_indices, var_node_name_indices, edge_index, variable_master_node_ids, type_emb, name_emb, W_hybrid, b_hybrid, W_msg, W_gru, U_gru, b_gru):
    raise NotImplementedError("write your pallas kernel here")


# speedup vs baseline: 1.8841x; 1.8841x over previous
import functools

import jax
import jax.numpy as jnp
from jax import lax
from jax.experimental import pallas as pl
from jax.experimental.pallas import tpu as pltpu
from jax.experimental.pallas import tpu_sc as plsc

H = 128
N = 10000
E = 320000
TIMESTEPS = 8

NC = 2
NS = 16
NW = NC * NS

NPAD = 10240
NPT = NPAD // NW
CHUNK = 128
NCHD = E // CHUNK
WROWS = NPT + 8

GATHER_ROWS = 2 * NPAD
GPT = GATHER_ROWS // NW
NVM = 2000
NVM_PAD = 2048
VPT = NVM_PAD // NW

_SC_PARAMS = pltpu.CompilerParams(needs_layout_passes=False)


def _sc_mesh():
  return plsc.VectorSubcoreMesh(core_axis_name="c", subcore_axis_name="s")


def _make_sc_gather(n_out, chunks, take):
  @functools.partial(
      pl.kernel,
      mesh=_sc_mesh(),
      out_type=jax.ShapeDtypeStruct((n_out, H), jnp.float32),
      scratch_types=[
          pltpu.VMEM((chunks, CHUNK), jnp.int32),
          pltpu.VMEM((chunks * CHUNK, H), jnp.float32),
          pltpu.SemaphoreType.DMA,
      ],
      compiler_params=_SC_PARAMS,
  )
  def k(table_hbm, idx_hbm, out_hbm, idx_v, rows_v, sem):
    c = lax.axis_index("c")
    s = lax.axis_index("s")
    wid = s * NC + c
    pltpu.sync_copy(idx_hbm.at[wid], idx_v)
    for j in range(chunks):
      pltpu.async_copy(
          table_hbm.at[idx_v.at[j]],
          rows_v.at[pl.ds(j * CHUNK, CHUNK)],
          sem,
      ).wait()
    pltpu.sync_copy(
        rows_v.at[pl.ds(0, take)], out_hbm.at[pl.ds(wid * take, take)]
    )

  return k


def _accum_chunk(o_ref, rows, win):
  def egbody(eg, carry):
    olv = o_ref[pl.ds(eg * 16, 16)]
    for l in range(16):
      r = olv[l]
      e = eg * 16 + l
      for k in range(8):
        plsc.addupdate(
            win.at[r, pl.ds(k * 16, 16)],
            rows[e, pl.ds(k * 16, 16)],
        )
    return carry

  lax.fori_loop(0, 8, egbody, 0)


def _sc_msg_body(ab_hbm, bnd_hbm, gch_hbm, och_hbm, bg_hbm, bo_hbm, zwin_hbm,
                 p_hbm, b_v, g_v, o_v, bga, boa, bgb, bob,
                 rows, win_a, win_b, sem):
  c = lax.axis_index("c")
  s = lax.axis_index("s")
  w = s * NC + c

  pltpu.sync_copy(zwin_hbm, win_a)
  pltpu.sync_copy(zwin_hbm, win_b)
  pltpu.sync_copy(bnd_hbm.at[w], b_v)
  bv = b_v[...]

  for d, win in ((0, win_a), (1, win_b)):
    c0 = bv[2 * d]
    c1 = bv[2 * d + 1]
    pltpu.sync_copy(bg_hbm.at[d, w, 0], bga)
    pltpu.sync_copy(bo_hbm.at[d, w, 0], boa)
    pltpu.sync_copy(bg_hbm.at[d, w, 1], bgb)
    pltpu.sync_copy(bo_hbm.at[d, w, 1], bob)

    pltpu.async_copy(ab_hbm.at[bga], rows, sem).wait()
    _accum_chunk(boa, rows, win)

    def body(j, carry, d=d, win=win):
      pltpu.sync_copy(gch_hbm.at[d, j], g_v)
      pltpu.sync_copy(och_hbm.at[d, j], o_v)
      pltpu.async_copy(ab_hbm.at[g_v], rows, sem).wait()
      _accum_chunk(o_v, rows, win)
      return carry

    lax.fori_loop(c0, c1, body, 0)

    pltpu.async_copy(ab_hbm.at[bgb], rows, sem).wait()
    _accum_chunk(bob, rows, win)

  def addrows(i, carry):
    for k in range(8):
      win_a[i, pl.ds(k * 16, 16)] = (
          win_a[i, pl.ds(k * 16, 16)] + win_b[i, pl.ds(k * 16, 16)])
    return carry

  lax.fori_loop(0, NPT, addrows, 0)

  pltpu.sync_copy(win_a.at[pl.ds(0, NPT)], p_hbm.at[pl.ds(w * NPT, NPT)])


def _make_sc_msg():
  return functools.partial(
      pl.kernel,
      mesh=_sc_mesh(),
      out_type=jax.ShapeDtypeStruct((NPAD, H), jnp.float32),
      scratch_types=[
          pltpu.VMEM((16,), jnp.int32),
          pltpu.VMEM((CHUNK,), jnp.int32),
          pltpu.VMEM((CHUNK,), jnp.int32),
          pltpu.VMEM((CHUNK,), jnp.int32),
          pltpu.VMEM((CHUNK,), jnp.int32),
          pltpu.VMEM((CHUNK,), jnp.int32),
          pltpu.VMEM((CHUNK,), jnp.int32),
          pltpu.VMEM((CHUNK, H), jnp.float32),
          pltpu.VMEM((WROWS, H), jnp.float32),
          pltpu.VMEM((WROWS, H), jnp.float32),
          pltpu.SemaphoreType.DMA,
      ],
      compiler_params=_SC_PARAMS,
  )(_sc_msg_body)


def _make_sc_gather_vm():
  @functools.partial(
      pl.kernel,
      mesh=_sc_mesh(),
      out_type=jax.ShapeDtypeStruct((NVM_PAD, H), jnp.float32),
      scratch_types=[
          pltpu.VMEM((1, VPT), jnp.int32),
          pltpu.VMEM((VPT, H), jnp.float32),
          pltpu.SemaphoreType.DMA,
      ],
      compiler_params=_SC_PARAMS,
  )
  def k(table_hbm, idx_hbm, out_hbm, idx_v, rows_v, sem):
    c = lax.axis_index("c")
    s = lax.axis_index("s")
    wid = s * NC + c
    pltpu.sync_copy(idx_hbm.at[wid], idx_v)
    pltpu.async_copy(table_hbm.at[idx_v.at[0]], rows_v, sem).wait()
    pltpu.sync_copy(rows_v, out_hbm.at[pl.ds(wid * VPT, VPT)])

  return k


RB = 1024


def _bdot(a, b):
  return jnp.dot(a.astype(jnp.bfloat16), b.astype(jnp.bfloat16),
                 preferred_element_type=jnp.float32)


def _tc_init_kernel(g_ref, wh_ref, bh_ref, wm_ref, h_ref, ab_ref):
  hcat = jnp.concatenate([g_ref[0], g_ref[1]], axis=1)
  h0 = _bdot(hcat, wh_ref[...]) + bh_ref[0:1, :]
  h_ref[...] = h0
  ab_ref[0] = _bdot(h0, wm_ref[0])
  ab_ref[1] = _bdot(h0, wm_ref[1])


def _tc_gru_kernel(p_ref, h_ref, wg_ref, ug_ref, bg_ref, wm_ref,
                   hn_ref, ab_ref):
  msg = p_ref[...]
  h = h_ref[...]
  gx = _bdot(msg, wg_ref[...]) + bg_ref[0:1, :]
  gh = _bdot(h, ug_ref[...])

  def sigmoid(x):
    return 1.0 / (1.0 + jnp.exp(-x))

  z = sigmoid(gx[:, :H] + gh[:, :H])
  r = sigmoid(gx[:, H:2 * H] + gh[:, H:2 * H])
  ht = jnp.tanh(gx[:, 2 * H:] + r * gh[:, 2 * H:])
  hn = (1.0 - z) * h + z * ht
  hn_ref[...] = hn
  ab_ref[0] = _bdot(hn, wm_ref[0])
  ab_ref[1] = _bdot(hn, wm_ref[1])


def _full(shape):
  return pl.BlockSpec(shape, lambda i: tuple(0 for _ in shape))


def _tc_init(g2, wh, bh, wm):
  return pl.pallas_call(
      _tc_init_kernel,
      grid=(NPAD // RB,),
      in_specs=[
          pl.BlockSpec((2, RB, H), lambda i: (0, i, 0)),
          _full((2 * H, H)),
          _full((8, H)),
          _full((2, H, H)),
      ],
      out_specs=[
          pl.BlockSpec((RB, H), lambda i: (i, 0)),
          pl.BlockSpec((2, RB, H), lambda i: (0, i, 0)),
      ],
      out_shape=[
          jax.ShapeDtypeStruct((NPAD, H), jnp.float32),
          jax.ShapeDtypeStruct((2, NPAD, H), jnp.float32),
      ],
      compiler_params=pltpu.CompilerParams(
          dimension_semantics=("arbitrary",)),
  )(g2, wh, bh, wm)


def _tc_gru(p, h, wg, ug, bg, wm):
  return pl.pallas_call(
      _tc_gru_kernel,
      grid=(NPAD // RB,),
      in_specs=[
          pl.BlockSpec((RB, H), lambda i: (i, 0)),
          pl.BlockSpec((RB, H), lambda i: (i, 0)),
          _full((H, 3 * H)),
          _full((H, 3 * H)),
          _full((8, 3 * H)),
          _full((2, H, H)),
      ],
      out_specs=[
          pl.BlockSpec((RB, H), lambda i: (i, 0)),
          pl.BlockSpec((2, RB, H), lambda i: (0, i, 0)),
      ],
      out_shape=[
          jax.ShapeDtypeStruct((NPAD, H), jnp.float32),
          jax.ShapeDtypeStruct((2, NPAD, H), jnp.float32),
      ],
      compiler_params=pltpu.CompilerParams(
          dimension_semantics=("arbitrary",)),
  )(p, h, wg, ug, bg, wm)


def _dir_slabs(g, o):
  perm = jnp.argsort(o // NPT, stable=True)
  gs = g[perm]
  osrt = o[perm]
  olocal = osrt % NPT
  bsort = osrt // NPT

  b = jnp.searchsorted(bsort, jnp.arange(NW + 1)).astype(jnp.int32)
  a_ch = b[:NW] // CHUNK
  b_ch = b[1:] // CHUNK
  c0 = a_ch + 1
  c1 = jnp.maximum(c0, b_ch)

  lane = jnp.arange(CHUNK, dtype=jnp.int32)

  def masked(ch, is_b):
    chc = jnp.minimum(ch, NCHD - 1)
    p = chc[:, None] * CHUNK + lane[None, :]
    valid = (p >= b[:NW, None]) & (p < b[1:, None])
    valid &= (p // CHUNK) == ch[:, None]
    if is_b:
      valid &= (b_ch != a_ch)[:, None]
    gm = jnp.where(valid, gs[p], 0)
    om = jnp.where(valid, olocal[p], NPT + (p & 7))
    return gm, om

  ga, oa = masked(a_ch, False)
  gb, ob = masked(b_ch, True)
  bg = jnp.stack([ga, gb], axis=1)
  bo = jnp.stack([oa, ob], axis=1)
  return (c0, c1, gs.reshape(NCHD, CHUNK), olocal.reshape(NCHD, CHUNK),
          bg, bo)


def _edge_slabs(src, dst):
  c0a, c1a, gch0, och0, bg0, bo0 = _dir_slabs(src, dst)
  c0b, c1b, gch1, och1, bg1, bo1 = _dir_slabs(dst + NPAD, src)
  bnd = jnp.zeros((NW, 16), jnp.int32)
  bnd = (bnd.at[:, 0].set(c0a).at[:, 1].set(c1a)
            .at[:, 2].set(c0b).at[:, 3].set(c1b))
  gch = jnp.stack([gch0, gch1])
  och = jnp.stack([och0, och1])
  bg = jnp.stack([bg0, bg1])
  bo = jnp.stack([bo0, bo1])
  return bnd, gch, och, bg, bo


def _scan_steps(step, init):
  carry, _ = lax.scan(step, init, None, length=TIMESTEPS)
  return carry


@jax.jit
def kernel(node_type_indices, var_node_name_indices, edge_index,
           variable_master_node_ids, type_emb, name_emb, W_hybrid, b_hybrid,
           W_msg, W_gru, U_gru, b_gru):
  nti = node_type_indices.astype(jnp.int32)
  vni = var_node_name_indices.astype(jnp.int32)
  src = edge_index[0].astype(jnp.int32)
  dst = edge_index[1].astype(jnp.int32)
  vmn = variable_master_node_ids.astype(jnp.int32)

  tbl = jnp.concatenate([type_emb, name_emb], axis=0)
  idx_init = jnp.zeros((GATHER_ROWS,), jnp.int32)
  idx_init = idx_init.at[:N].set(nti)
  idx_init = idx_init.at[NPAD:NPAD + N].set(vni + type_emb.shape[0])
  idx_init = idx_init.reshape(NW, GPT // CHUNK, CHUNK)
  g_flat = _make_sc_gather(GATHER_ROWS, GPT // CHUNK, GPT)(tbl, idx_init)
  g2 = g_flat.reshape(2, NPAD, H)

  bh = jnp.broadcast_to(b_hybrid, (8, H))
  bg_gru = jnp.broadcast_to(b_gru, (8, 3 * H))
  h0, ab = _tc_init(g2, W_hybrid, bh, W_msg)

  bnd, gch, och, bg, bo = _edge_slabs(src, dst)
  zwin = jnp.zeros((WROWS, H), jnp.float32)

  sc_msg = _make_sc_msg()

  def step(carry, _):
    h, ab = carry
    p = sc_msg(ab.reshape(2 * NPAD, H), bnd, gch, och, bg, bo, zwin)
    h2, ab2 = _tc_gru(p, h, W_gru, U_gru, bg_gru, W_msg)
    return (h2, ab2), None

  h, ab = _scan_steps(step, (h0, ab))

  idx_vm = jnp.zeros((NVM_PAD,), jnp.int32).at[:NVM].set(vmn)
  idx_vm = idx_vm.reshape(NW, 1, VPT)
  vm = _make_sc_gather_vm()(h, idx_vm)

  return (h0[:N], h[:N], vm[:NVM])
